# baseline (speedup 1.0000x reference)
import functools

import numpy as np
import jax
import jax.numpy as jnp
from jax import lax
from jax.experimental import pallas as pl
from jax.experimental.pallas import tpu as pltpu
from jax.experimental.pallas import tpu_sc as plsc

_N_LEVELS = 16
_LEVEL_DIM = 2
_LOG2_T = 19
_T = 2 ** _LOG2_T
_BASE_RES = 16
_DESIRED_RES = 4096
_SCALE = float(np.exp2(np.log2(_DESIRED_RES / _BASE_RES) / (_N_LEVELS - 1)))
_RES = [int(np.floor(_BASE_RES * _SCALE ** l)) for l in range(_N_LEVELS)]
_P1 = int(np.uint32(2654435761).astype(np.int32))
_P2 = int(np.uint32(805459861).astype(np.int32))
_MASK = _T - 1
_HI = int(np.uint32(0xFFFF0000).astype(np.int32))

_NC, _NS = 2, 16
_NW = _NC * _NS
_B = 1024
_GRP = 512
_G = _B // _GRP
_SSL = _T // _NS
_NHC = 6


def _encode_body(xf, tpk, resb, grid, pxyz, wb, idxb, rwb, levb, resv, spm,
                 gsems, hsems, stsem, osems):
    n = xf.shape[0] // 3
    ppw = n // _NW
    nch = ppw // _B
    cid = lax.axis_index("c")
    sid = lax.axis_index("s")
    wid = sid * _NC + cid
    wbase = wid * ppw

    pltpu.sync_copy(resb, resv)
    for d in range(3):
        pltpu.sync_copy(xf.at[pl.ds(d * n + wbase, ppw)], pxyz[d])

    def sig_body(i, _):
        off = i * 16
        for d in range(3):
            v = pxyz[d][pl.ds(off, 16)]
            pxyz[d][pl.ds(off, 16)] = 1.0 / (1.0 + jnp.exp(-2.0 * v))
        return 0

    lax.fori_loop(0, ppw // 16, sig_body, 0)

    def pass1(l, pc, coff):
        resvec = resv[l, pl.ds(0, 16)]
        lofs = jnp.full((16,), l * _T, jnp.int32)

        def body(i, _):
            off = i * 16
            posx = pxyz[0][pl.ds(coff + off, 16)] * resvec
            posy = pxyz[1][pl.ds(coff + off, 16)] * resvec
            posz = pxyz[2][pl.ds(coff + off, 16)] * resvec
            ix = posx.astype(jnp.int32)
            iy = posy.astype(jnp.int32)
            iz = posz.astype(jnp.int32)
            wb[pc][0][pl.ds(off, 16)] = posx - ix.astype(jnp.float32)
            wb[pc][1][pl.ds(off, 16)] = posy - iy.astype(jnp.float32)
            wb[pc][2][pl.ds(off, 16)] = posz - iz.astype(jnp.float32)
            hx = (ix, ix + 1)
            hy0 = iy * _P1
            hy = (hy0, hy0 + _P1)
            hz0 = iz * _P2
            hz = (hz0, hz0 + _P2)
            for dz in range(2):
                for dy in range(2):
                    t = hy[dy] ^ hz[dz]
                    for dx in range(2):
                        c = dx + 2 * dy + 4 * dz
                        h = (hx[dx] ^ t) & _MASK
                        idxb[pc][c][pl.ds(off, 16)] = (
                            h if c < _NHC else h + lofs)
            return 0

        lax.fori_loop(0, _B // 16, body, 0)

    def fire(pc):
        for gi in range(_G):
            for c in range(8):
                src = spm if c < _NHC else tpk
                sem = gsems[pc] if c < _NHC else hsems[pc]
                pltpu.async_copy(
                    src.at[idxb[pc][c].at[pl.ds(gi * _GRP, _GRP)]],
                    rwb[pc][c].at[pl.ds(gi * _GRP, _GRP)],
                    sem,
                )

    def drain(pc):
        for gi in range(_G):
            for c in range(8):
                src = spm if c < _NHC else tpk
                sem = gsems[pc] if c < _NHC else hsems[pc]
                pltpu.make_async_copy(
                    src.at[idxb[pc][c].at[pl.ds(gi * _GRP, _GRP)]],
                    rwb[pc][c].at[pl.ds(gi * _GRP, _GRP)],
                    sem,
                ).wait()

    def out_copy(l, q, coff):
        for f in range(2):
            pltpu.async_copy(
                levb[q][f],
                grid.at[pl.ds((2 * l + f) * n + wbase + coff, _B)],
                osems[q],
            )

    def out_drain(q):
        for f in range(2):
            pltpu.make_async_copy(
                levb[q][f],
                grid.at[pl.ds(f * n, _B)],
                osems[q],
            ).wait()

    def pass2(l, pc, coff):
        def body(i, _):
            off = i * 16
            wx = wb[pc][0][pl.ds(off, 16)]
            wy = wb[pc][1][pl.ds(off, 16)]
            wz = wb[pc][2][pl.ds(off, 16)]
            ex = (1.0 - wx, wx)
            ey = (1.0 - wy, wy)
            ez = (1.0 - wz, wz)
            u = [[ey[dy] * ez[dz] for dz in range(2)] for dy in range(2)]
            acc0 = jnp.zeros((16,), jnp.float32)
            acc1 = jnp.zeros((16,), jnp.float32)
            for dz in range(2):
                for dy in range(2):
                    for dx in range(2):
                        c = dx + 2 * dy + 4 * dz
                        wgt = ex[dx] * u[dy][dz]
                        pk = rwb[pc][c][pl.ds(off, 16)]
                        f0 = lax.bitcast_convert_type(pk & _HI, jnp.float32)
                        f1 = lax.bitcast_convert_type(pk << 16, jnp.float32)
                        acc0 = acc0 + wgt * f0
                        acc1 = acc1 + wgt * f1
            levb[pc][0][pl.ds(off, 16)] = acc0
            levb[pc][1][pl.ds(off, 16)] = acc1
            return 0

        lax.fori_loop(0, _B // 16, body, 0)
        out_copy(l, pc, coff)

    def level_body(l, _):
        plsc.subcore_barrier()
        pltpu.async_copy(
            tpk.at[pl.ds(l * _T + sid * _SSL, _SSL)],
            spm.at[pl.ds(sid * _SSL, _SSL)],
            stsem,
        )
        pltpu.make_async_copy(
            tpk.at[pl.ds(0, _SSL)],
            spm.at[pl.ds(sid * _SSL, _SSL)],
            stsem,
        ).wait()
        plsc.subcore_barrier()

        pass1(l, 0, 0)
        fire(0)

        def cpair(cp, _):
            a2 = 2 * cp

            pass1(l, 1, (a2 + 1) * _B)
            fire(1)

            @pl.when(cp > 0)
            def _():
                out_drain(0)

            drain(0)
            pass2(l, 0, a2 * _B)

            @pl.when(cp < nch // 2 - 1)
            def _():
                pass1(l, 0, (a2 + 2) * _B)
                fire(0)

            @pl.when(cp > 0)
            def _():
                out_drain(1)

            drain(1)
            pass2(l, 1, (a2 + 1) * _B)
            return 0

        lax.fori_loop(0, nch // 2, cpair, 0)
        out_drain(0)
        out_drain(1)
        return 0

    lax.fori_loop(0, _N_LEVELS, level_body, 0)


def _encode(xf, tpk, resb):
    n = xf.shape[0] // 3
    ppw = n // _NW

    def body(xf_r, tpk_r, resb_r, grid_r, *s):
        pxyz = s[0:3]
        wb = (s[3:6], s[6:9])
        idxb = (s[9:17], s[17:25])
        rwb = (s[25:33], s[33:41])
        levb = (s[41:43], s[43:45])
        resv = s[45]
        spm = s[46]
        gsems = s[47:49]
        hsems = s[49:51]
        stsem = s[51]
        osems = s[52:54]
        _encode_body(xf_r, tpk_r, resb_r, grid_r, pxyz, wb, idxb, rwb,
                     levb, resv, spm, gsems, hsems, stsem, osems)

    return pl.kernel(
        body,
        out_type=jax.ShapeDtypeStruct((2 * _N_LEVELS * n,), jnp.float32),
        mesh=plsc.VectorSubcoreMesh(core_axis_name="c", subcore_axis_name="s"),
        scratch_types=(
            [pltpu.VMEM((ppw,), jnp.float32) for _ in range(3)]
            + [pltpu.VMEM((_B,), jnp.float32) for _ in range(6)]
            + [pltpu.VMEM((_B,), jnp.int32) for _ in range(16)]
            + [pltpu.VMEM((_B,), jnp.int32) for _ in range(16)]
            + [pltpu.VMEM((_B,), jnp.float32) for _ in range(4)]
            + [pltpu.VMEM((16, 16), jnp.float32)]
            + [pltpu.VMEM_SHARED((_T,), jnp.int32)]
            + [pltpu.SemaphoreType.DMA] * 7
        ),
    )(xf, tpk, resb)


_BT = 4096


def _mlp_body(g_ref, w1_ref, b1_ref, w2t_ref, b2_ref, o_ref):
    h = jnp.dot(
        w1_ref[...], g_ref[...],
        preferred_element_type=jnp.float32,
        precision=lax.Precision.HIGHEST,
    )
    h = jnp.maximum(h + b1_ref[...], 0.0)
    o_ref[...] = jnp.sum(h * w2t_ref[...], axis=0, keepdims=True) + b2_ref[...]


def _mlp(grid_t, W1, b1, W2t, b2):
    n = grid_t.shape[1]
    gd = grid_t.shape[0]
    hid = W1.shape[0]
    return pl.pallas_call(
        _mlp_body,
        grid=(n // _BT,),
        in_specs=[
            pl.BlockSpec((gd, _BT), lambda j: (0, j)),
            pl.BlockSpec((hid, gd), lambda j: (0, 0)),
            pl.BlockSpec((hid, 1), lambda j: (0, 0)),
            pl.BlockSpec((hid, 1), lambda j: (0, 0)),
            pl.BlockSpec((1, 1), lambda j: (0, 0)),
        ],
        out_specs=pl.BlockSpec((1, _BT), lambda j: (0, j)),
        out_shape=jax.ShapeDtypeStruct((1, n), jnp.float32),
    )(grid_t, W1, b1.reshape(hid, 1), W2t, b2.reshape(1, 1))


def kernel(x, table, W1, b1, W2, b2):
    n = x.shape[0]
    xf = x.T.reshape(-1)
    tb = table.astype(jnp.bfloat16)
    hi = lax.bitcast_convert_type(tb[:, :, 0], jnp.uint16).astype(jnp.uint32)
    lo = lax.bitcast_convert_type(tb[:, :, 1], jnp.uint16).astype(jnp.uint32)
    tpk = lax.bitcast_convert_type((hi << 16) | lo, jnp.int32).reshape(-1)
    resb = jnp.tile(
        jnp.asarray(_RES, dtype=jnp.float32)[:, None], (1, 16)
    )
    grid_t = _encode(xf, tpk, resb).reshape(2 * _N_LEVELS, n)
    out = _mlp(grid_t, W1, b1, W2.reshape(-1, 1), b2)
    return out.reshape(n, 1)


# speedup vs baseline: 1.0815x; 1.0815x over previous
import functools

import numpy as np
import jax
import jax.numpy as jnp
from jax import lax
from jax.experimental import pallas as pl
from jax.experimental.pallas import tpu as pltpu
from jax.experimental.pallas import tpu_sc as plsc

_N_LEVELS = 16
_LEVEL_DIM = 2
_LOG2_T = 19
_T = 2 ** _LOG2_T
_BASE_RES = 16
_DESIRED_RES = 4096
_SCALE = float(np.exp2(np.log2(_DESIRED_RES / _BASE_RES) / (_N_LEVELS - 1)))
_RES = [int(np.floor(_BASE_RES * _SCALE ** l)) for l in range(_N_LEVELS)]
_P1 = int(np.uint32(2654435761).astype(np.int32))
_P2 = int(np.uint32(805459861).astype(np.int32))
_MASK = _T - 1
_HI = int(np.uint32(0xFFFF0000).astype(np.int32))

_NC, _NS = 2, 16
_NW = _NC * _NS
_B = 1024
_GRP = 512
_G = _B // _GRP
_SSL = _T // _NS


def _encode_body(xf, tpk, resb, grid, pxyz, idxb, rwb, levb, resv, spm,
                 gsems, stsems, osems):
    n = xf.shape[0] // 3
    ppw = n // _NW
    nch = ppw // _B
    cid = lax.axis_index("c")
    sid = lax.axis_index("s")
    wid = sid * _NC + cid
    wbase = wid * ppw

    pltpu.sync_copy(resb, resv)
    for d in range(3):
        pltpu.sync_copy(xf.at[pl.ds(d * n + wbase, ppw)], pxyz[d])

    def sig_body(i, _):
        off = i * 16
        for d in range(3):
            v = pxyz[d][pl.ds(off, 16)]
            pxyz[d][pl.ds(off, 16)] = 1.0 / (1.0 + jnp.exp(-2.0 * v))
        return 0

    lax.fori_loop(0, ppw // 16, sig_body, 0)

    def stage_fire(lvl, spar):
        pltpu.async_copy(
            tpk.at[pl.ds(lvl * _T + sid * _SSL, _SSL)],
            spm.at[pl.ds(spar * _T + sid * _SSL, _SSL)],
            stsems[spar],
        )

    def stage_wait(spar):
        pltpu.make_async_copy(
            tpk.at[pl.ds(0, _SSL)],
            spm.at[pl.ds(spar * _T + sid * _SSL, _SSL)],
            stsems[spar],
        ).wait()

    def pass1(l, pc, coff, loff):
        resvec = resv[l, pl.ds(0, 16)]

        def body(i, _):
            off = i * 16
            posx = pxyz[0][pl.ds(coff + off, 16)] * resvec
            posy = pxyz[1][pl.ds(coff + off, 16)] * resvec
            posz = pxyz[2][pl.ds(coff + off, 16)] * resvec
            ix = posx.astype(jnp.int32)
            iy = posy.astype(jnp.int32)
            iz = posz.astype(jnp.int32)
            hx = (ix, ix + 1)
            hy0 = iy * _P1
            hy = (hy0, hy0 + _P1)
            hz0 = iz * _P2
            hz = (hz0, hz0 + _P2)
            for dz in range(2):
                for dy in range(2):
                    t = hy[dy] ^ hz[dz]
                    for dx in range(2):
                        c = dx + 2 * dy + 4 * dz
                        h = (hx[dx] ^ t) & _MASK
                        idxb[pc][c][pl.ds(off, 16)] = h + loff if loff else h
            return 0

        lax.fori_loop(0, _B // 16, body, 0)

    def fire(pc):
        for gi in range(_G):
            for c in range(8):
                pltpu.async_copy(
                    spm.at[idxb[pc][c].at[pl.ds(gi * _GRP, _GRP)]],
                    rwb[pc][c].at[pl.ds(gi * _GRP, _GRP)],
                    gsems[pc],
                )

    def drain(pc):
        for gi in range(_G):
            for c in range(8):
                pltpu.make_async_copy(
                    spm.at[idxb[pc][c].at[pl.ds(gi * _GRP, _GRP)]],
                    rwb[pc][c].at[pl.ds(gi * _GRP, _GRP)],
                    gsems[pc],
                ).wait()

    def out_copy(l, q, coff):
        for f in range(2):
            pltpu.async_copy(
                levb[q][f],
                grid.at[pl.ds((2 * l + f) * n + wbase + coff, _B)],
                osems[q],
            )

    def out_drain(q):
        for f in range(2):
            pltpu.make_async_copy(
                levb[q][f],
                grid.at[pl.ds(f * n, _B)],
                osems[q],
            ).wait()

    def pass2(l, pc, coff):
        resvec = resv[l, pl.ds(0, 16)]

        def body(i, _):
            off = i * 16
            posx = pxyz[0][pl.ds(coff + off, 16)] * resvec
            posy = pxyz[1][pl.ds(coff + off, 16)] * resvec
            posz = pxyz[2][pl.ds(coff + off, 16)] * resvec
            wx = posx - posx.astype(jnp.int32).astype(jnp.float32)
            wy = posy - posy.astype(jnp.int32).astype(jnp.float32)
            wz = posz - posz.astype(jnp.int32).astype(jnp.float32)
            ex = (1.0 - wx, wx)
            ey = (1.0 - wy, wy)
            ez = (1.0 - wz, wz)
            u = [[ey[dy] * ez[dz] for dz in range(2)] for dy in range(2)]
            acc0 = jnp.zeros((16,), jnp.float32)
            acc1 = jnp.zeros((16,), jnp.float32)
            for dz in range(2):
                for dy in range(2):
                    for dx in range(2):
                        c = dx + 2 * dy + 4 * dz
                        wgt = ex[dx] * u[dy][dz]
                        pk = rwb[pc][c][pl.ds(off, 16)]
                        f0 = lax.bitcast_convert_type(pk & _HI, jnp.float32)
                        f1 = lax.bitcast_convert_type(pk << 16, jnp.float32)
                        acc0 = acc0 + wgt * f0
                        acc1 = acc1 + wgt * f1
            levb[pc][0][pl.ds(off, 16)] = acc0
            levb[pc][1][pl.ds(off, 16)] = acc1
            return 0

        lax.fori_loop(0, _B // 16, body, 0)
        out_copy(l, pc, coff)

    def proc_level(l, loff):
        pass1(l, 0, 0, loff)
        fire(0)

        def cpair(cp, _):
            a2 = 2 * cp

            pass1(l, 1, (a2 + 1) * _B, loff)
            fire(1)

            @pl.when(cp > 0)
            def _():
                out_drain(0)

            drain(0)
            pass2(l, 0, a2 * _B)

            @pl.when(cp < nch // 2 - 1)
            def _():
                pass1(l, 0, (a2 + 2) * _B, loff)
                fire(0)

            @pl.when(cp > 0)
            def _():
                out_drain(1)

            drain(1)
            pass2(l, 1, (a2 + 1) * _B)
            return 0

        lax.fori_loop(0, nch // 2, cpair, 0)
        out_drain(0)
        out_drain(1)

    stage_fire(0, 0)

    def lpair(l2, _):
        a = 2 * l2
        b = a + 1

        stage_fire(b, 1)
        stage_wait(0)
        plsc.subcore_barrier()
        proc_level(a, 0)
        plsc.subcore_barrier()

        @pl.when(l2 < _N_LEVELS // 2 - 1)
        def _():
            stage_fire(a + 2, 0)

        stage_wait(1)
        plsc.subcore_barrier()
        proc_level(b, _T)
        plsc.subcore_barrier()
        return 0

    lax.fori_loop(0, _N_LEVELS // 2, lpair, 0)


def _encode(xf, tpk, resb):
    n = xf.shape[0] // 3
    ppw = n // _NW

    def body(xf_r, tpk_r, resb_r, grid_r, *s):
        pxyz = s[0:3]
        idxb = (s[3:11], s[11:19])
        rwb = (s[19:27], s[27:35])
        levb = (s[35:37], s[37:39])
        resv = s[39]
        spm = s[40]
        gsems = s[41:43]
        stsems = s[43:45]
        osems = s[45:47]
        _encode_body(xf_r, tpk_r, resb_r, grid_r, pxyz, idxb, rwb,
                     levb, resv, spm, gsems, stsems, osems)

    return pl.kernel(
        body,
        out_type=jax.ShapeDtypeStruct((2 * _N_LEVELS * n,), jnp.float32),
        mesh=plsc.VectorSubcoreMesh(core_axis_name="c", subcore_axis_name="s"),
        scratch_types=(
            [pltpu.VMEM((ppw,), jnp.float32) for _ in range(3)]
            + [pltpu.VMEM((_B,), jnp.int32) for _ in range(16)]
            + [pltpu.VMEM((_B,), jnp.int32) for _ in range(16)]
            + [pltpu.VMEM((_B,), jnp.float32) for _ in range(4)]
            + [pltpu.VMEM((16, 16), jnp.float32)]
            + [pltpu.VMEM_SHARED((2 * _T,), jnp.int32)]
            + [pltpu.SemaphoreType.DMA] * 6
        ),
    )(xf, tpk, resb)


_BT = 4096


def _mlp_body(g_ref, w1_ref, b1_ref, w2t_ref, b2_ref, o_ref):
    h = jnp.dot(
        w1_ref[...], g_ref[...],
        preferred_element_type=jnp.float32,
        precision=lax.Precision.HIGHEST,
    )
    h = jnp.maximum(h + b1_ref[...], 0.0)
    o_ref[...] = jnp.sum(h * w2t_ref[...], axis=0, keepdims=True) + b2_ref[...]


def _mlp(grid_t, W1, b1, W2t, b2):
    n = grid_t.shape[1]
    gd = grid_t.shape[0]
    hid = W1.shape[0]
    return pl.pallas_call(
        _mlp_body,
        grid=(n // _BT,),
        in_specs=[
            pl.BlockSpec((gd, _BT), lambda j: (0, j)),
            pl.BlockSpec((hid, gd), lambda j: (0, 0)),
            pl.BlockSpec((hid, 1), lambda j: (0, 0)),
            pl.BlockSpec((hid, 1), lambda j: (0, 0)),
            pl.BlockSpec((1, 1), lambda j: (0, 0)),
        ],
        out_specs=pl.BlockSpec((1, _BT), lambda j: (0, j)),
        out_shape=jax.ShapeDtypeStruct((1, n), jnp.float32),
    )(grid_t, W1, b1.reshape(hid, 1), W2t, b2.reshape(1, 1))


def kernel(x, table, W1, b1, W2, b2):
    n = x.shape[0]
    xf = x.T.reshape(-1)
    tb = table.astype(jnp.bfloat16)
    hi = lax.bitcast_convert_type(tb[:, :, 0], jnp.uint16).astype(jnp.uint32)
    lo = lax.bitcast_convert_type(tb[:, :, 1], jnp.uint16).astype(jnp.uint32)
    tpk = lax.bitcast_convert_type((hi << 16) | lo, jnp.int32).reshape(-1)
    resb = jnp.tile(
        jnp.asarray(_RES, dtype=jnp.float32)[:, None], (1, 16)
    )
    grid_t = _encode(xf, tpk, resb).reshape(2 * _N_LEVELS, n)
    out = _mlp(grid_t, W1, b1, W2.reshape(-1, 1), b2)
    return out.reshape(n, 1)


# speedup vs baseline: 1.0965x; 1.0139x over previous
import functools

import numpy as np
import jax
import jax.numpy as jnp
from jax import lax
from jax.experimental import pallas as pl
from jax.experimental.pallas import tpu as pltpu
from jax.experimental.pallas import tpu_sc as plsc

_N_LEVELS = 16
_LEVEL_DIM = 2
_LOG2_T = 19
_T = 2 ** _LOG2_T
_BASE_RES = 16
_DESIRED_RES = 4096
_SCALE = float(np.exp2(np.log2(_DESIRED_RES / _BASE_RES) / (_N_LEVELS - 1)))
_RES = [int(np.floor(_BASE_RES * _SCALE ** l)) for l in range(_N_LEVELS)]
_P1 = int(np.uint32(2654435761).astype(np.int32))
_P2 = int(np.uint32(805459861).astype(np.int32))
_MASK = _T - 1
_HI = int(np.uint32(0xFFFF0000).astype(np.int32))

_NC, _NS = 2, 16
_NW = _NC * _NS
_B = 1024
_GRP = 512
_G = _B // _GRP
_SSL = _T // _NS


def _encode_body(xf, tpk, resb, grid, pxyz, wb, idxb, rwb, levb, resv, spm,
                 gsems, stsem, osems):
    n = xf.shape[0] // 3
    ppw = n // _NW
    nch = ppw // _B
    cid = lax.axis_index("c")
    sid = lax.axis_index("s")
    wid = sid * _NC + cid
    wbase = wid * ppw

    pltpu.sync_copy(resb, resv)
    for d in range(3):
        pltpu.sync_copy(xf.at[pl.ds(d * n + wbase, ppw)], pxyz[d])

    def sig_body(i, _):
        off = i * 16
        for d in range(3):
            v = pxyz[d][pl.ds(off, 16)]
            pxyz[d][pl.ds(off, 16)] = 1.0 / (1.0 + jnp.exp(-2.0 * v))
        return 0

    lax.fori_loop(0, ppw // 16, sig_body, 0)

    def pass1(l, pc, coff):
        resvec = resv[l, pl.ds(0, 16)]

        def body(i, _):
            off = i * 16
            posx = pxyz[0][pl.ds(coff + off, 16)] * resvec
            posy = pxyz[1][pl.ds(coff + off, 16)] * resvec
            posz = pxyz[2][pl.ds(coff + off, 16)] * resvec
            ix = posx.astype(jnp.int32)
            iy = posy.astype(jnp.int32)
            iz = posz.astype(jnp.int32)
            wb[pc][0][pl.ds(off, 16)] = posx - ix.astype(jnp.float32)
            wb[pc][1][pl.ds(off, 16)] = posy - iy.astype(jnp.float32)
            wb[pc][2][pl.ds(off, 16)] = posz - iz.astype(jnp.float32)
            hx = (ix, ix + 1)
            hy0 = iy * _P1
            hy = (hy0, hy0 + _P1)
            hz0 = iz * _P2
            hz = (hz0, hz0 + _P2)
            for dz in range(2):
                for dy in range(2):
                    t = hy[dy] ^ hz[dz]
                    for dx in range(2):
                        c = dx + 2 * dy + 4 * dz
                        idxb[pc][c][pl.ds(off, 16)] = (hx[dx] ^ t) & _MASK
            return 0

        lax.fori_loop(0, _B // 16, body, 0)

    def fire(pc):
        for gi in range(_G):
            for c in range(8):
                pltpu.async_copy(
                    spm.at[idxb[pc][c].at[pl.ds(gi * _GRP, _GRP)]],
                    rwb[pc][c].at[pl.ds(gi * _GRP, _GRP)],
                    gsems[pc],
                )

    def drain(pc):
        for gi in range(_G):
            for c in range(8):
                pltpu.make_async_copy(
                    spm.at[idxb[pc][c].at[pl.ds(gi * _GRP, _GRP)]],
                    rwb[pc][c].at[pl.ds(gi * _GRP, _GRP)],
                    gsems[pc],
                ).wait()

    def out_copy(l, q, coff):
        for f in range(2):
            pltpu.async_copy(
                levb[q][f],
                grid.at[pl.ds((2 * l + f) * n + wbase + coff, _B)],
                osems[q],
            )

    def out_drain(q):
        for f in range(2):
            pltpu.make_async_copy(
                levb[q][f],
                grid.at[pl.ds(f * n, _B)],
                osems[q],
            ).wait()

    def pass2(l, pc, coff):
        def body(i, _):
            off = i * 16
            wx = wb[pc][0][pl.ds(off, 16)]
            wy = wb[pc][1][pl.ds(off, 16)]
            wz = wb[pc][2][pl.ds(off, 16)]
            ex = (1.0 - wx, wx)
            ey = (1.0 - wy, wy)
            ez = (1.0 - wz, wz)
            u = [[ey[dy] * ez[dz] for dz in range(2)] for dy in range(2)]
            acc0 = jnp.zeros((16,), jnp.float32)
            acc1 = jnp.zeros((16,), jnp.float32)
            for dz in range(2):
                for dy in range(2):
                    for dx in range(2):
                        c = dx + 2 * dy + 4 * dz
                        wgt = ex[dx] * u[dy][dz]
                        pk = rwb[pc][c][pl.ds(off, 16)]
                        f0 = lax.bitcast_convert_type(pk & _HI, jnp.float32)
                        f1 = lax.bitcast_convert_type(pk << 16, jnp.float32)
                        acc0 = acc0 + wgt * f0
                        acc1 = acc1 + wgt * f1
            levb[pc][0][pl.ds(off, 16)] = acc0
            levb[pc][1][pl.ds(off, 16)] = acc1
            return 0

        lax.fori_loop(0, _B // 16, body, 0)
        out_copy(l, pc, coff)

    def level_body(l, _):
        plsc.subcore_barrier()
        pltpu.async_copy(
            tpk.at[pl.ds(l * _T + sid * _SSL, _SSL)],
            spm.at[pl.ds(sid * _SSL, _SSL)],
            stsem,
        )
        pltpu.make_async_copy(
            tpk.at[pl.ds(0, _SSL)],
            spm.at[pl.ds(sid * _SSL, _SSL)],
            stsem,
        ).wait()
        plsc.subcore_barrier()

        pass1(l, 0, 0)
        fire(0)

        def cpair(cp, _):
            a2 = 2 * cp

            pass1(l, 1, (a2 + 1) * _B)
            fire(1)

            @pl.when(cp > 0)
            def _():
                out_drain(0)

            drain(0)
            pass2(l, 0, a2 * _B)

            @pl.when(cp < nch // 2 - 1)
            def _():
                pass1(l, 0, (a2 + 2) * _B)
                fire(0)

            @pl.when(cp > 0)
            def _():
                out_drain(1)

            drain(1)
            pass2(l, 1, (a2 + 1) * _B)
            return 0

        lax.fori_loop(0, nch // 2, cpair, 0)
        out_drain(0)
        out_drain(1)
        return 0

    lax.fori_loop(0, _N_LEVELS, level_body, 0)


def _encode(xf, tpk, resb):
    n = xf.shape[0] // 3
    ppw = n // _NW

    def body(xf_r, tpk_r, resb_r, grid_r, *s):
        pxyz = s[0:3]
        wb = (s[3:6], s[6:9])
        idxb = (s[9:17], s[17:25])
        rwb = (s[25:33], s[33:41])
        levb = (s[41:43], s[43:45])
        resv = s[45]
        spm = s[46]
        gsems = s[47:49]
        stsem = s[49]
        osems = s[50:52]
        _encode_body(xf_r, tpk_r, resb_r, grid_r, pxyz, wb, idxb, rwb,
                     levb, resv, spm, gsems, stsem, osems)

    return pl.kernel(
        body,
        out_type=jax.ShapeDtypeStruct((2 * _N_LEVELS * n,), jnp.float32),
        mesh=plsc.VectorSubcoreMesh(core_axis_name="c", subcore_axis_name="s"),
        scratch_types=(
            [pltpu.VMEM((ppw,), jnp.float32) for _ in range(3)]
            + [pltpu.VMEM((_B,), jnp.float32) for _ in range(6)]
            + [pltpu.VMEM((_B,), jnp.int32) for _ in range(16)]
            + [pltpu.VMEM((_B,), jnp.int32) for _ in range(16)]
            + [pltpu.VMEM((_B,), jnp.float32) for _ in range(4)]
            + [pltpu.VMEM((16, 16), jnp.float32)]
            + [pltpu.VMEM_SHARED((_T,), jnp.int32)]
            + [pltpu.SemaphoreType.DMA] * 5
        ),
    )(xf, tpk, resb)


_BT = 4096


def _mlp_body(g_ref, w1_ref, b1_ref, w2t_ref, b2_ref, o_ref):
    h = jnp.dot(
        w1_ref[...], g_ref[...],
        preferred_element_type=jnp.float32,
        precision=lax.Precision.HIGHEST,
    )
    h = jnp.maximum(h + b1_ref[...], 0.0)
    o_ref[...] = jnp.sum(h * w2t_ref[...], axis=0, keepdims=True) + b2_ref[...]


def _mlp(grid_t, W1, b1, W2t, b2):
    n = grid_t.shape[1]
    gd = grid_t.shape[0]
    hid = W1.shape[0]
    return pl.pallas_call(
        _mlp_body,
        grid=(n // _BT,),
        in_specs=[
            pl.BlockSpec((gd, _BT), lambda j: (0, j)),
            pl.BlockSpec((hid, gd), lambda j: (0, 0)),
            pl.BlockSpec((hid, 1), lambda j: (0, 0)),
            pl.BlockSpec((hid, 1), lambda j: (0, 0)),
            pl.BlockSpec((1, 1), lambda j: (0, 0)),
        ],
        out_specs=pl.BlockSpec((1, _BT), lambda j: (0, j)),
        out_shape=jax.ShapeDtypeStruct((1, n), jnp.float32),
    )(grid_t, W1, b1.reshape(hid, 1), W2t, b2.reshape(1, 1))


def kernel(x, table, W1, b1, W2, b2):
    n = x.shape[0]
    xf = x.T.reshape(-1)
    tb = table.astype(jnp.bfloat16)
    hi = lax.bitcast_convert_type(tb[:, :, 0], jnp.uint16).astype(jnp.uint32)
    lo = lax.bitcast_convert_type(tb[:, :, 1], jnp.uint16).astype(jnp.uint32)
    tpk = lax.bitcast_convert_type((hi << 16) | lo, jnp.int32).reshape(-1)
    resb = jnp.tile(
        jnp.asarray(_RES, dtype=jnp.float32)[:, None], (1, 16)
    )
    grid_t = _encode(xf, tpk, resb).reshape(2 * _N_LEVELS, n)
    out = _mlp(grid_t, W1, b1, W2.reshape(-1, 1), b2)
    return out.reshape(n, 1)


# speedup vs baseline: 1.2063x; 1.1001x over previous
import functools

import numpy as np
import jax
import jax.numpy as jnp
from jax import lax
from jax.experimental import pallas as pl
from jax.experimental.pallas import tpu as pltpu
from jax.experimental.pallas import tpu_sc as plsc

_N_LEVELS = 16
_LEVEL_DIM = 2
_LOG2_T = 19
_T = 2 ** _LOG2_T
_BASE_RES = 16
_DESIRED_RES = 4096
_SCALE = float(np.exp2(np.log2(_DESIRED_RES / _BASE_RES) / (_N_LEVELS - 1)))
_RES = [int(np.floor(_BASE_RES * _SCALE ** l)) for l in range(_N_LEVELS)]
_P1 = int(np.uint32(2654435761).astype(np.int32))
_P2 = int(np.uint32(805459861).astype(np.int32))
_MASK = _T - 1
_HI = int(np.uint32(0xFFFF0000).astype(np.int32))

_NC, _NS = 2, 16
_NW = _NC * _NS
_B = 512
_GRP = 512
_G = _B // _GRP
_SSL = _T // _NS


def _encode_body(xf, tpk, resb, grid, pxyz, wb, idxb, rwb, levb, resv, spms,
                 gsems, stsems, osems):
    n = xf.shape[0] // 3
    ppw = n // _NW
    nch = ppw // _B
    cid = lax.axis_index("c")
    sid = lax.axis_index("s")
    wid = sid * _NC + cid
    wbase = wid * ppw

    def stage(l, q):
        pltpu.async_copy(
            tpk.at[pl.ds(l * _T + sid * _SSL, _SSL)],
            spms[q].at[pl.ds(sid * _SSL, _SSL)],
            stsems[q],
        )

    def stage_wait(q):
        pltpu.make_async_copy(
            tpk.at[pl.ds(0, _SSL)],
            spms[q].at[pl.ds(sid * _SSL, _SSL)],
            stsems[q],
        ).wait()

    stage(0, 0)

    pltpu.sync_copy(resb, resv)
    for d in range(3):
        pltpu.sync_copy(xf.at[pl.ds(d * n + wbase, ppw)], pxyz[d])

    def sig_body(i, _):
        off = i * 16
        for d in range(3):
            v = pxyz[d][pl.ds(off, 16)]
            pxyz[d][pl.ds(off, 16)] = 1.0 / (1.0 + jnp.exp(-2.0 * v))
        return 0

    lax.fori_loop(0, ppw // 16, sig_body, 0)

    def pass1(l, pc, coff):
        resvec = resv[l, pl.ds(0, 16)]

        def body(i, _):
            off = i * 16
            posx = pxyz[0][pl.ds(coff + off, 16)] * resvec
            posy = pxyz[1][pl.ds(coff + off, 16)] * resvec
            posz = pxyz[2][pl.ds(coff + off, 16)] * resvec
            ix = posx.astype(jnp.int32)
            iy = posy.astype(jnp.int32)
            iz = posz.astype(jnp.int32)
            wb[pc][0][pl.ds(off, 16)] = posx - ix.astype(jnp.float32)
            wb[pc][1][pl.ds(off, 16)] = posy - iy.astype(jnp.float32)
            wb[pc][2][pl.ds(off, 16)] = posz - iz.astype(jnp.float32)
            hx = (ix, ix + 1)
            hy0 = iy * _P1
            hy = (hy0, hy0 + _P1)
            hz0 = iz * _P2
            hz = (hz0, hz0 + _P2)
            for dz in range(2):
                for dy in range(2):
                    t = hy[dy] ^ hz[dz]
                    for dx in range(2):
                        c = dx + 2 * dy + 4 * dz
                        idxb[pc][c][pl.ds(off, 16)] = (hx[dx] ^ t) & _MASK
            return 0

        lax.fori_loop(0, _B // 16, body, 0)

    def fire(pc, spm):
        for gi in range(_G):
            for c in range(8):
                pltpu.async_copy(
                    spm.at[idxb[pc][c].at[pl.ds(gi * _GRP, _GRP)]],
                    rwb[pc][c].at[pl.ds(gi * _GRP, _GRP)],
                    gsems[pc],
                )

    def drain(pc, spm):
        for gi in range(_G):
            for c in range(8):
                pltpu.make_async_copy(
                    spm.at[idxb[pc][c].at[pl.ds(gi * _GRP, _GRP)]],
                    rwb[pc][c].at[pl.ds(gi * _GRP, _GRP)],
                    gsems[pc],
                ).wait()

    def out_copy(l, q, coff):
        for f in range(2):
            pltpu.async_copy(
                levb[q][f],
                grid.at[pl.ds((2 * l + f) * n + wbase + coff, _B)],
                osems[q],
            )

    def out_drain(q):
        for f in range(2):
            pltpu.make_async_copy(
                levb[q][f],
                grid.at[pl.ds(f * n, _B)],
                osems[q],
            ).wait()

    def pass2(l, pc, coff):
        def body(i, _):
            off = i * 16
            wx = wb[pc][0][pl.ds(off, 16)]
            wy = wb[pc][1][pl.ds(off, 16)]
            wz = wb[pc][2][pl.ds(off, 16)]
            ex = (1.0 - wx, wx)
            ey = (1.0 - wy, wy)
            ez = (1.0 - wz, wz)
            u = [[ey[dy] * ez[dz] for dz in range(2)] for dy in range(2)]
            acc0 = jnp.zeros((16,), jnp.float32)
            acc1 = jnp.zeros((16,), jnp.float32)
            for dz in range(2):
                for dy in range(2):
                    for dx in range(2):
                        c = dx + 2 * dy + 4 * dz
                        wgt = ex[dx] * u[dy][dz]
                        pk = rwb[pc][c][pl.ds(off, 16)]
                        f0 = lax.bitcast_convert_type(pk & _HI, jnp.float32)
                        f1 = lax.bitcast_convert_type(pk << 16, jnp.float32)
                        acc0 = acc0 + wgt * f0
                        acc1 = acc1 + wgt * f1
            levb[pc][0][pl.ds(off, 16)] = acc0
            levb[pc][1][pl.ds(off, 16)] = acc1
            return 0

        lax.fori_loop(0, _B // 16, body, 0)
        out_copy(l, pc, coff)

    def run_level(l, spm):
        pass1(l, 0, 0)
        fire(0, spm)

        def cpair(cp, _):
            a2 = 2 * cp

            pass1(l, 1, (a2 + 1) * _B)
            fire(1, spm)

            @pl.when(cp > 0)
            def _():
                out_drain(0)

            drain(0, spm)
            pass2(l, 0, a2 * _B)

            @pl.when(cp < nch // 2 - 1)
            def _():
                pass1(l, 0, (a2 + 2) * _B)
                fire(0, spm)

            @pl.when(cp > 0)
            def _():
                out_drain(1)

            drain(1, spm)
            pass2(l, 1, (a2 + 1) * _B)
            return 0

        lax.fori_loop(0, nch // 2, cpair, 0)
        out_drain(0)
        out_drain(1)

    def level_pair(p, _):
        l0 = 2 * p
        l1 = 2 * p + 1

        plsc.subcore_barrier()
        stage(l1, 1)
        stage_wait(0)
        plsc.subcore_barrier()
        run_level(l0, spms[0])

        plsc.subcore_barrier()

        @pl.when(p < _N_LEVELS // 2 - 1)
        def _():
            stage(l1 + 1, 0)

        stage_wait(1)
        plsc.subcore_barrier()
        run_level(l1, spms[1])
        return 0

    lax.fori_loop(0, _N_LEVELS // 2, level_pair, 0)


def _encode(xf, tpk, resb):
    n = xf.shape[0] // 3
    ppw = n // _NW

    def body(xf_r, tpk_r, resb_r, grid_r, *s):
        pxyz = s[0:3]
        wb = (s[3:6], s[6:9])
        idxb = (s[9:17], s[17:25])
        rwb = (s[25:33], s[33:41])
        levb = (s[41:43], s[43:45])
        resv = s[45]
        spms = s[46:48]
        gsems = s[48:50]
        stsems = s[50:52]
        osems = s[52:54]
        _encode_body(xf_r, tpk_r, resb_r, grid_r, pxyz, wb, idxb, rwb,
                     levb, resv, spms, gsems, stsems, osems)

    return pl.kernel(
        body,
        out_type=jax.ShapeDtypeStruct((2 * _N_LEVELS * n,), jnp.float32),
        mesh=plsc.VectorSubcoreMesh(core_axis_name="c", subcore_axis_name="s"),
        scratch_types=(
            [pltpu.VMEM((ppw,), jnp.float32) for _ in range(3)]
            + [pltpu.VMEM((_B,), jnp.float32) for _ in range(6)]
            + [pltpu.VMEM((_B,), jnp.int32) for _ in range(16)]
            + [pltpu.VMEM((_B,), jnp.int32) for _ in range(16)]
            + [pltpu.VMEM((_B,), jnp.float32) for _ in range(4)]
            + [pltpu.VMEM((16, 16), jnp.float32)]
            + [pltpu.VMEM_SHARED((_T,), jnp.int32)] * 2
            + [pltpu.SemaphoreType.DMA] * 6
        ),
    )(xf, tpk, resb)


_BT = 4096


def _mlp_body(g_ref, w1_ref, b1_ref, w2t_ref, b2_ref, o_ref):
    h = jnp.dot(
        w1_ref[...], g_ref[...],
        preferred_element_type=jnp.float32,
        precision=lax.Precision.HIGHEST,
    )
    h = jnp.maximum(h + b1_ref[...], 0.0)
    o_ref[...] = jnp.sum(h * w2t_ref[...], axis=0, keepdims=True) + b2_ref[...]


def _mlp(grid_t, W1, b1, W2t, b2):
    n = grid_t.shape[1]
    gd = grid_t.shape[0]
    hid = W1.shape[0]
    return pl.pallas_call(
        _mlp_body,
        grid=(n // _BT,),
        in_specs=[
            pl.BlockSpec((gd, _BT), lambda j: (0, j)),
            pl.BlockSpec((hid, gd), lambda j: (0, 0)),
            pl.BlockSpec((hid, 1), lambda j: (0, 0)),
            pl.BlockSpec((hid, 1), lambda j: (0, 0)),
            pl.BlockSpec((1, 1), lambda j: (0, 0)),
        ],
        out_specs=pl.BlockSpec((1, _BT), lambda j: (0, j)),
        out_shape=jax.ShapeDtypeStruct((1, n), jnp.float32),
    )(grid_t, W1, b1.reshape(hid, 1), W2t, b2.reshape(1, 1))


def kernel(x, table, W1, b1, W2, b2):
    n = x.shape[0]
    xf = x.T.reshape(-1)
    tb = table.astype(jnp.bfloat16)
    hi = lax.bitcast_convert_type(tb[:, :, 0], jnp.uint16).astype(jnp.uint32)
    lo = lax.bitcast_convert_type(tb[:, :, 1], jnp.uint16).astype(jnp.uint32)
    tpk = lax.bitcast_convert_type((hi << 16) | lo, jnp.int32).reshape(-1)
    resb = jnp.tile(
        jnp.asarray(_RES, dtype=jnp.float32)[:, None], (1, 16)
    )
    grid_t = _encode(xf, tpk, resb).reshape(2 * _N_LEVELS, n)
    out = _mlp(grid_t, W1, b1, W2.reshape(-1, 1), b2)
    return out.reshape(n, 1)


# speedup vs baseline: 1.2206x; 1.0119x over previous
import functools

import numpy as np
import jax
import jax.numpy as jnp
from jax import lax
from jax.experimental import pallas as pl
from jax.experimental.pallas import tpu as pltpu
from jax.experimental.pallas import tpu_sc as plsc

_N_LEVELS = 16
_LEVEL_DIM = 2
_LOG2_T = 19
_T = 2 ** _LOG2_T
_BASE_RES = 16
_DESIRED_RES = 4096
_SCALE = float(np.exp2(np.log2(_DESIRED_RES / _BASE_RES) / (_N_LEVELS - 1)))
_RES = [int(np.floor(_BASE_RES * _SCALE ** l)) for l in range(_N_LEVELS)]
_P1 = int(np.uint32(2654435761).astype(np.int32))
_P2 = int(np.uint32(805459861).astype(np.int32))
_MASK = _T - 1
_HI = int(np.uint32(0xFFFF0000).astype(np.int32))

_NC, _NS = 2, 16
_NW = _NC * _NS
_B = 512
_GRP = 512
_G = _B // _GRP
_SSL = _T // _NS


def _encode_body(xf, tpk, resb, grid, pxyz, wb, idxb, rwb, levb, resv, spms,
                 gsems, stsems, osems):
    n = xf.shape[0] // 3
    ppw = n // _NW
    nch = ppw // _B
    cid = lax.axis_index("c")
    sid = lax.axis_index("s")
    wid = sid * _NC + cid
    wbase = wid * ppw

    def stage(l, q):
        pltpu.async_copy(
            tpk.at[pl.ds(l * _T + sid * _SSL, _SSL)],
            spms[q].at[pl.ds(sid * _SSL, _SSL)],
            stsems[q],
        )

    def stage_wait(q):
        pltpu.make_async_copy(
            tpk.at[pl.ds(0, _SSL)],
            spms[q].at[pl.ds(sid * _SSL, _SSL)],
            stsems[q],
        ).wait()

    stage(0, 0)

    pltpu.sync_copy(resb, resv)
    for d in range(3):
        pltpu.sync_copy(xf.at[pl.ds(d * n + wbase, ppw)], pxyz[d])

    def sig_body(i, _):
        off = i * 16
        for d in range(3):
            v = pxyz[d][pl.ds(off, 16)]
            pxyz[d][pl.ds(off, 16)] = 1.0 / (1.0 + jnp.exp(-2.0 * v))
        return 0

    lax.fori_loop(0, ppw // 16, sig_body, 0)

    def pass1(l, pc, coff):
        resvec = resv[l, pl.ds(0, 16)]

        def body(i, _):
            off = i * 16
            posx = pxyz[0][pl.ds(coff + off, 16)] * resvec
            posy = pxyz[1][pl.ds(coff + off, 16)] * resvec
            posz = pxyz[2][pl.ds(coff + off, 16)] * resvec
            ix = posx.astype(jnp.int32)
            iy = posy.astype(jnp.int32)
            iz = posz.astype(jnp.int32)
            wb[pc][0][pl.ds(off, 16)] = posx - ix.astype(jnp.float32)
            wb[pc][1][pl.ds(off, 16)] = posy - iy.astype(jnp.float32)
            wb[pc][2][pl.ds(off, 16)] = posz - iz.astype(jnp.float32)
            hx = (ix, ix + 1)
            hy0 = iy * _P1
            hy = (hy0, hy0 + _P1)
            hz0 = iz * _P2
            hz = (hz0, hz0 + _P2)
            for dz in range(2):
                for dy in range(2):
                    t = hy[dy] ^ hz[dz]
                    for dx in range(2):
                        c = dx + 2 * dy + 4 * dz
                        idxb[pc][c][pl.ds(off, 16)] = (hx[dx] ^ t) & _MASK
            return 0

        lax.fori_loop(0, _B // 16, body, 0)

    def fire(pc, spm):
        for gi in range(_G):
            for c in range(8):
                pltpu.async_copy(
                    spm.at[idxb[pc][c].at[pl.ds(gi * _GRP, _GRP)]],
                    rwb[pc][c].at[pl.ds(gi * _GRP, _GRP)],
                    gsems[pc],
                )

    def drain(pc, spm):
        for gi in range(_G):
            for c in range(8):
                pltpu.make_async_copy(
                    spm.at[idxb[pc][c].at[pl.ds(gi * _GRP, _GRP)]],
                    rwb[pc][c].at[pl.ds(gi * _GRP, _GRP)],
                    gsems[pc],
                ).wait()

    def out_copy(l, q, coff):
        for f in range(2):
            pltpu.async_copy(
                levb[q][f],
                grid.at[pl.ds((2 * l + f) * n + wbase + coff, _B)],
                osems[q],
            )

    def out_drain(q):
        for f in range(2):
            pltpu.make_async_copy(
                levb[q][f],
                grid.at[pl.ds(f * n, _B)],
                osems[q],
            ).wait()

    def pass2(l, pc, coff):
        def body(i, _):
            off = i * 16
            wx = wb[pc][0][pl.ds(off, 16)]
            wy = wb[pc][1][pl.ds(off, 16)]
            wz = wb[pc][2][pl.ds(off, 16)]
            ex = (1.0 - wx, wx)
            ey = (1.0 - wy, wy)
            ez = (1.0 - wz, wz)
            u = [[ey[dy] * ez[dz] for dz in range(2)] for dy in range(2)]
            acc0 = jnp.zeros((16,), jnp.float32)
            acc1 = jnp.zeros((16,), jnp.float32)
            for dz in range(2):
                for dy in range(2):
                    for dx in range(2):
                        c = dx + 2 * dy + 4 * dz
                        wgt = ex[dx] * u[dy][dz]
                        pk = rwb[pc][c][pl.ds(off, 16)]
                        f0 = lax.bitcast_convert_type(pk & _HI, jnp.float32)
                        f1 = lax.bitcast_convert_type(pk << 16, jnp.float32)
                        acc0 = acc0 + wgt * f0
                        acc1 = acc1 + wgt * f1
            levb[pc][0][pl.ds(off, 16)] = acc0
            levb[pc][1][pl.ds(off, 16)] = acc1
            return 0

        lax.fori_loop(0, _B // 16, body, 0)
        out_copy(l, pc, coff)

    def run_level(l, spm):
        pass1(l, 0, 0)
        fire(0, spm)

        def cpair(cp, _):
            a2 = 2 * cp

            pass1(l, 1, (a2 + 1) * _B)
            fire(1, spm)

            @pl.when(cp > 0)
            def _():
                out_drain(0)

            drain(0, spm)
            pass2(l, 0, a2 * _B)

            @pl.when(cp < nch // 2 - 1)
            def _():
                pass1(l, 0, (a2 + 2) * _B)
                fire(0, spm)

            @pl.when(cp > 0)
            def _():
                out_drain(1)

            drain(1, spm)
            pass2(l, 1, (a2 + 1) * _B)
            return 0

        lax.fori_loop(0, nch // 2, cpair, 0)
        out_drain(0)
        out_drain(1)

    def level_pair(p, _):
        l0 = 2 * p
        l1 = 2 * p + 1

        plsc.subcore_barrier()
        stage(l1, 1)
        stage_wait(0)
        plsc.subcore_barrier()
        run_level(l0, spms[0])

        plsc.subcore_barrier()

        @pl.when(p < _N_LEVELS // 2 - 1)
        def _():
            stage(l1 + 1, 0)

        stage_wait(1)
        plsc.subcore_barrier()
        run_level(l1, spms[1])
        return 0

    lax.fori_loop(0, _N_LEVELS // 2, level_pair, 0)


def _encode(xf, tpk, resb):
    n = xf.shape[0] // 3
    ppw = n // _NW

    def body(xf_r, tpk_r, resb_r, grid_r, *s):
        pxyz = s[0:3]
        wb = (s[3:6], s[6:9])
        idxb = (s[9:17], s[17:25])
        rwb = (s[25:33], s[33:41])
        levb = (s[41:43], s[43:45])
        resv = s[45]
        spms = s[46:48]
        gsems = s[48:50]
        stsems = s[50:52]
        osems = s[52:54]
        _encode_body(xf_r, tpk_r, resb_r, grid_r, pxyz, wb, idxb, rwb,
                     levb, resv, spms, gsems, stsems, osems)

    return pl.kernel(
        body,
        out_type=jax.ShapeDtypeStruct((2 * _N_LEVELS * n,), jnp.float32),
        mesh=plsc.VectorSubcoreMesh(core_axis_name="c", subcore_axis_name="s"),
        scratch_types=(
            [pltpu.VMEM((ppw,), jnp.float32) for _ in range(3)]
            + [pltpu.VMEM((_B,), jnp.float32) for _ in range(6)]
            + [pltpu.VMEM((_B,), jnp.int32) for _ in range(16)]
            + [pltpu.VMEM((_B,), jnp.int32) for _ in range(16)]
            + [pltpu.VMEM((_B,), jnp.float32) for _ in range(4)]
            + [pltpu.VMEM((16, 16), jnp.float32)]
            + [pltpu.VMEM_SHARED((_T,), jnp.int32)] * 2
            + [pltpu.SemaphoreType.DMA] * 6
        ),
    )(xf, tpk, resb)


_BT = 4096


def _mlp_body(g_ref, w1_ref, b1_ref, w2t_ref, b2_ref, o_ref):
    h = jnp.dot(
        w1_ref[...], g_ref[...],
        preferred_element_type=jnp.float32,
        precision=lax.Precision.HIGHEST,
    )
    h = jnp.maximum(h + b1_ref[...], 0.0)
    o_ref[...] = jnp.sum(h * w2t_ref[...], axis=0, keepdims=True) + b2_ref[...]


def _mlp(grid_t, W1, b1, W2t, b2):
    n = grid_t.shape[1]
    gd = grid_t.shape[0]
    hid = W1.shape[0]
    return pl.pallas_call(
        _mlp_body,
        grid=(n // _BT,),
        in_specs=[
            pl.BlockSpec((gd, _BT), lambda j: (0, j)),
            pl.BlockSpec((hid, gd), lambda j: (0, 0)),
            pl.BlockSpec((hid, 1), lambda j: (0, 0)),
            pl.BlockSpec((hid, 1), lambda j: (0, 0)),
            pl.BlockSpec((1, 1), lambda j: (0, 0)),
        ],
        out_specs=pl.BlockSpec((1, _BT), lambda j: (0, j)),
        out_shape=jax.ShapeDtypeStruct((1, n), jnp.float32),
    )(grid_t, W1, b1.reshape(hid, 1), W2t, b2.reshape(1, 1))


def kernel(x, table, W1, b1, W2, b2):
    n = x.shape[0]
    tb = table.astype(jnp.bfloat16)
    hi = lax.bitcast_convert_type(tb[:, :, 0], jnp.uint16).astype(jnp.uint32)
    lo = lax.bitcast_convert_type(tb[:, :, 1], jnp.uint16).astype(jnp.uint32)
    tpk = lax.bitcast_convert_type((hi << 16) | lo, jnp.int32).reshape(-1)
    resb = jnp.tile(
        jnp.asarray(_RES, dtype=jnp.float32)[:, None], (1, 16)
    )
    h = n // 2
    W2c = W2.reshape(-1, 1)
    outs = []
    for half in (x[:h], x[h:]):
        xf = half.T.reshape(-1)
        grid_t = _encode(xf, tpk, resb).reshape(2 * _N_LEVELS, h)
        outs.append(_mlp(grid_t, W1, b1, W2c, b2))
    return jnp.concatenate(outs, axis=1).reshape(n, 1)


# speedup vs baseline: 1.2435x; 1.0188x over previous
import functools

import numpy as np
import jax
import jax.numpy as jnp
from jax import lax
from jax.experimental import pallas as pl
from jax.experimental.pallas import tpu as pltpu
from jax.experimental.pallas import tpu_sc as plsc

_N_LEVELS = 16
_LEVEL_DIM = 2
_LOG2_T = 19
_T = 2 ** _LOG2_T
_BASE_RES = 16
_DESIRED_RES = 4096
_SCALE = float(np.exp2(np.log2(_DESIRED_RES / _BASE_RES) / (_N_LEVELS - 1)))
_RES = [int(np.floor(_BASE_RES * _SCALE ** l)) for l in range(_N_LEVELS)]
_P1 = int(np.uint32(2654435761).astype(np.int32))
_P2 = int(np.uint32(805459861).astype(np.int32))
_MASK = _T - 1
_HI = int(np.uint32(0xFFFF0000).astype(np.int32))

_NC, _NS = 2, 16
_NW = _NC * _NS
_B = 512
_GRP = 512
_G = _B // _GRP
_SSL = _T // _NS


def _encode_body(xf, tpk, resb, grid, pxyz, wb, idxb, rwb, levb, resv, spms,
                 gsems, stsems, osems):
    n = xf.shape[0] // 3
    ppw = n // _NW
    nch = ppw // _B
    cid = lax.axis_index("c")
    sid = lax.axis_index("s")
    wid = sid * _NC + cid
    wbase = wid * ppw

    def stage(l, q):
        pltpu.async_copy(
            tpk.at[pl.ds(l * _T + sid * _SSL, _SSL)],
            spms[q].at[pl.ds(sid * _SSL, _SSL)],
            stsems[q],
        )

    def stage_wait(q):
        pltpu.make_async_copy(
            tpk.at[pl.ds(0, _SSL)],
            spms[q].at[pl.ds(sid * _SSL, _SSL)],
            stsems[q],
        ).wait()

    stage(0, 0)

    pltpu.sync_copy(resb, resv)
    for d in range(3):
        pltpu.sync_copy(xf.at[pl.ds(d * n + wbase, ppw)], pxyz[d])

    def sig_body(i, _):
        off = i * 16
        for d in range(3):
            v = pxyz[d][pl.ds(off, 16)]
            pxyz[d][pl.ds(off, 16)] = 1.0 / (1.0 + jnp.exp(-2.0 * v))
        return 0

    lax.fori_loop(0, ppw // 16, sig_body, 0)

    def pass1(l, pc, coff):
        resvec = resv[l, pl.ds(0, 16)]

        def body(i, _):
            off = i * 16
            posx = pxyz[0][pl.ds(coff + off, 16)] * resvec
            posy = pxyz[1][pl.ds(coff + off, 16)] * resvec
            posz = pxyz[2][pl.ds(coff + off, 16)] * resvec
            ix = posx.astype(jnp.int32)
            iy = posy.astype(jnp.int32)
            iz = posz.astype(jnp.int32)
            wb[pc][0][pl.ds(off, 16)] = posx - ix.astype(jnp.float32)
            wb[pc][1][pl.ds(off, 16)] = posy - iy.astype(jnp.float32)
            wb[pc][2][pl.ds(off, 16)] = posz - iz.astype(jnp.float32)
            hx = (ix, ix + 1)
            hy0 = iy * _P1
            hy = (hy0, hy0 + _P1)
            hz0 = iz * _P2
            hz = (hz0, hz0 + _P2)
            for dz in range(2):
                for dy in range(2):
                    t = hy[dy] ^ hz[dz]
                    for dx in range(2):
                        c = dx + 2 * dy + 4 * dz
                        idxb[pc][c][pl.ds(off, 16)] = (hx[dx] ^ t) & _MASK
            return 0

        lax.fori_loop(0, _B // 16, body, 0)

    def fire(pc, spm):
        for gi in range(_G):
            for c in range(8):
                pltpu.async_copy(
                    spm.at[idxb[pc][c].at[pl.ds(gi * _GRP, _GRP)]],
                    rwb[pc][c].at[pl.ds(gi * _GRP, _GRP)],
                    gsems[pc],
                )

    def drain(pc, spm):
        for gi in range(_G):
            for c in range(8):
                pltpu.make_async_copy(
                    spm.at[idxb[pc][c].at[pl.ds(gi * _GRP, _GRP)]],
                    rwb[pc][c].at[pl.ds(gi * _GRP, _GRP)],
                    gsems[pc],
                ).wait()

    def out_copy(l, q, coff):
        for f in range(2):
            pltpu.async_copy(
                levb[q][f],
                grid.at[pl.ds((2 * l + f) * n + wbase + coff, _B)],
                osems[q],
            )

    def out_drain(q):
        for f in range(2):
            pltpu.make_async_copy(
                levb[q][f],
                grid.at[pl.ds(f * n, _B)],
                osems[q],
            ).wait()

    def pass2(l, pc, coff):
        def body(i, _):
            off = i * 16
            wx = wb[pc][0][pl.ds(off, 16)]
            wy = wb[pc][1][pl.ds(off, 16)]
            wz = wb[pc][2][pl.ds(off, 16)]
            ex = (1.0 - wx, wx)
            ey = (1.0 - wy, wy)
            ez = (1.0 - wz, wz)
            u = [[ey[dy] * ez[dz] for dz in range(2)] for dy in range(2)]
            acc0 = jnp.zeros((16,), jnp.float32)
            acc1 = jnp.zeros((16,), jnp.float32)
            for dz in range(2):
                for dy in range(2):
                    for dx in range(2):
                        c = dx + 2 * dy + 4 * dz
                        wgt = ex[dx] * u[dy][dz]
                        pk = rwb[pc][c][pl.ds(off, 16)]
                        f0 = lax.bitcast_convert_type(pk & _HI, jnp.float32)
                        f1 = lax.bitcast_convert_type(pk << 16, jnp.float32)
                        acc0 = acc0 + wgt * f0
                        acc1 = acc1 + wgt * f1
            levb[pc][0][pl.ds(off, 16)] = acc0
            levb[pc][1][pl.ds(off, 16)] = acc1
            return 0

        lax.fori_loop(0, _B // 16, body, 0)
        out_copy(l, pc, coff)

    def run_level(l, spm):
        pass1(l, 0, 0)
        fire(0, spm)

        def cpair(cp, _):
            a2 = 2 * cp

            pass1(l, 1, (a2 + 1) * _B)
            fire(1, spm)

            @pl.when(cp > 0)
            def _():
                out_drain(0)

            drain(0, spm)
            pass2(l, 0, a2 * _B)

            @pl.when(cp < nch // 2 - 1)
            def _():
                pass1(l, 0, (a2 + 2) * _B)
                fire(0, spm)

            @pl.when(cp > 0)
            def _():
                out_drain(1)

            drain(1, spm)
            pass2(l, 1, (a2 + 1) * _B)
            return 0

        lax.fori_loop(0, nch // 2, cpair, 0)
        out_drain(0)
        out_drain(1)

    def level_pair(p, _):
        l0 = 2 * p
        l1 = 2 * p + 1

        plsc.subcore_barrier()
        stage(l1, 1)
        stage_wait(0)
        plsc.subcore_barrier()
        run_level(l0, spms[0])

        plsc.subcore_barrier()

        @pl.when(p < _N_LEVELS // 2 - 1)
        def _():
            stage(l1 + 1, 0)

        stage_wait(1)
        plsc.subcore_barrier()
        run_level(l1, spms[1])
        return 0

    lax.fori_loop(0, _N_LEVELS // 2, level_pair, 0)


def _encode(xf, tpk, resb):
    n = xf.shape[0] // 3
    ppw = n // _NW

    def body(xf_r, tpk_r, resb_r, grid_r, *s):
        pxyz = s[0:3]
        wb = (s[3:6], s[6:9])
        idxb = (s[9:17], s[17:25])
        rwb = (s[25:33], s[33:41])
        levb = (s[41:43], s[43:45])
        resv = s[45]
        spms = s[46:48]
        gsems = s[48:50]
        stsems = s[50:52]
        osems = s[52:54]
        _encode_body(xf_r, tpk_r, resb_r, grid_r, pxyz, wb, idxb, rwb,
                     levb, resv, spms, gsems, stsems, osems)

    return pl.kernel(
        body,
        out_type=jax.ShapeDtypeStruct((2 * _N_LEVELS * n,), jnp.float32),
        mesh=plsc.VectorSubcoreMesh(core_axis_name="c", subcore_axis_name="s"),
        scratch_types=(
            [pltpu.VMEM((ppw,), jnp.float32) for _ in range(3)]
            + [pltpu.VMEM((_B,), jnp.float32) for _ in range(6)]
            + [pltpu.VMEM((_B,), jnp.int32) for _ in range(16)]
            + [pltpu.VMEM((_B,), jnp.int32) for _ in range(16)]
            + [pltpu.VMEM((_B,), jnp.float32) for _ in range(4)]
            + [pltpu.VMEM((16, 16), jnp.float32)]
            + [pltpu.VMEM_SHARED((_T,), jnp.int32)] * 2
            + [pltpu.SemaphoreType.DMA] * 6
        ),
    )(xf, tpk, resb)


_BT = 16384


def _mlp_body(g_ref, w1_ref, b1_ref, w2t_ref, b2_ref, o_ref):
    h = jnp.dot(
        w1_ref[...], g_ref[...],
        preferred_element_type=jnp.float32,
        precision=lax.Precision.HIGHEST,
    )
    h = jnp.maximum(h + b1_ref[...], 0.0)
    o_ref[...] = jnp.sum(h * w2t_ref[...], axis=0, keepdims=True) + b2_ref[...]


def _mlp(grid_t, W1, b1, W2t, b2):
    n = grid_t.shape[1]
    gd = grid_t.shape[0]
    hid = W1.shape[0]
    return pl.pallas_call(
        _mlp_body,
        grid=(n // _BT,),
        in_specs=[
            pl.BlockSpec((gd, _BT), lambda j: (0, j)),
            pl.BlockSpec((hid, gd), lambda j: (0, 0)),
            pl.BlockSpec((hid, 1), lambda j: (0, 0)),
            pl.BlockSpec((hid, 1), lambda j: (0, 0)),
            pl.BlockSpec((1, 1), lambda j: (0, 0)),
        ],
        out_specs=pl.BlockSpec((1, _BT), lambda j: (0, j)),
        out_shape=jax.ShapeDtypeStruct((1, n), jnp.float32),
    )(grid_t, W1, b1.reshape(hid, 1), W2t, b2.reshape(1, 1))


def kernel(x, table, W1, b1, W2, b2):
    n = x.shape[0]
    tb = table.astype(jnp.bfloat16)
    hi = lax.bitcast_convert_type(tb[:, :, 0], jnp.uint16).astype(jnp.uint32)
    lo = lax.bitcast_convert_type(tb[:, :, 1], jnp.uint16).astype(jnp.uint32)
    tpk = lax.bitcast_convert_type((hi << 16) | lo, jnp.int32).reshape(-1)
    resb = jnp.tile(
        jnp.asarray(_RES, dtype=jnp.float32)[:, None], (1, 16)
    )
    h = n // 2
    W2c = W2.reshape(-1, 1)
    outs = []
    for half in (x[:h], x[h:]):
        xf = half.T.reshape(-1)
        grid_t = _encode(xf, tpk, resb).reshape(2 * _N_LEVELS, h)
        outs.append(_mlp(grid_t, W1, b1, W2c, b2))
    return jnp.concatenate(outs, axis=1).reshape(n, 1)
